# baseline (speedup 1.0000x reference)
import functools

import jax
import jax.numpy as jnp
from jax import lax
from jax.experimental import pallas as pl
from jax.experimental.pallas import tpu as pltpu
from jax.experimental.pallas import tpu_sc as plsc

N = 10000
E = 320000
D = 128
DEPTH = 10
TELEPORTS = (0.1, 0.2, 0.3)

NW = 32
N_PAD = 10240
NB = N_PAD // NW
C = 128
S = 4096
EMAX = 16384
EP = E + S + C
RC = 64


def _sc_step_call(x, srcp, dstp, estp, maskp, invp):
    mesh = plsc.VectorSubcoreMesh(
        core_axis_name="c", subcore_axis_name="s", num_cores=2, num_subcores=16
    )

    @functools.partial(
        pl.kernel,
        out_type=jax.ShapeDtypeStruct((N_PAD, D), jnp.float32),
        mesh=mesh,
        compiler_params=pltpu.CompilerParams(needs_layout_passes=False),
        scratch_types=[
            pltpu.VMEM((NB + 8, D), jnp.float32),
            pltpu.VMEM((2, C, D), jnp.float32),
            pltpu.VMEM((EMAX,), jnp.int32),
            pltpu.VMEM((EMAX,), jnp.int32),
            pltpu.VMEM((48,), jnp.int32),
            pltpu.VMEM((NB,), jnp.float32),
            pltpu.VMEM((NB,), jnp.float32),
            pltpu.SemaphoreType.DMA,
            pltpu.SemaphoreType.DMA,
        ],
    )
    def step(x_hbm, srcp_hbm, dstp_hbm, est_hbm, mask_hbm, inv_hbm, y_hbm,
             acc, gbuf, sbuf, dbuf, est_v, mask_v, inv_v, sem0, sem1):
        cid = lax.axis_index("c")
        sid = lax.axis_index("s")
        wid = cid * 16 + sid
        nbase = wid * NB

        zero16 = jnp.zeros((16,), jnp.float32)

        def zrow(r, _):
            for j in range(D // 16):
                acc[r, pl.ds(j * 16, 16)] = zero16
            return 0

        lax.fori_loop(0, NB + 8, zrow, 0)

        pltpu.sync_copy(est_hbm, est_v)
        ew = est_v[pl.ds(wid, 16)]
        e0 = ew[0]
        e1 = ew[1]
        e0a = (e0 // 8) * 8
        nsb = (e1 - e0a + (EMAX - 1)) // EMAX

        cols = [lax.iota(jnp.int32, 16) + 16 * j for j in range(D // 16)]
        sems = (sem0, sem1)

        def accumulate(ci, buf):
            def egroup(g, _):
                dv = dbuf[pl.ds(ci * C + g * 16, 16)]
                locv16 = dv - nbase
                bad = jnp.logical_or(locv16 < 0, locv16 >= NB)
                locv16 = jnp.where(bad, NB, locv16)
                lo0 = locv16[0]
                lo15 = locv16[15]

                @pl.when(lo0 == lo15)
                def _():
                    for j in range(D // 16):
                        t = gbuf[buf, g * 16, pl.ds(j * 16, 16)]
                        for u in range(1, 16):
                            t = t + gbuf[buf, g * 16 + u, pl.ds(j * 16, 16)]
                        plsc.addupdate(acc.at[lo0, pl.ds(j * 16, 16)], t)

                @pl.when(lo0 != lo15)
                def _():
                    for h in range(2):
                        ha = locv16[h * 8]
                        hb = locv16[h * 8 + 7]

                        @pl.when(ha == hb)
                        def _():
                            for j in range(D // 16):
                                t = gbuf[buf, g * 16 + h * 8, pl.ds(j * 16, 16)]
                                for u in range(1, 8):
                                    t = t + gbuf[buf, g * 16 + h * 8 + u,
                                                 pl.ds(j * 16, 16)]
                                plsc.addupdate(acc.at[ha, pl.ds(j * 16, 16)], t)

                        @pl.when(ha != hb)
                        def _():
                            for u in range(8):
                                loc = locv16[h * 8 + u]
                                e = g * 16 + h * 8 + u
                                for j in range(D // 16):
                                    v = gbuf[buf, e, pl.ds(j * 16, 16)]
                                    plsc.addupdate(acc.at[loc, pl.ds(j * 16, 16)], v)
                return 0

            lax.fori_loop(0, C // 16, egroup, 0)

        def superblock(sb, _):
            sb_e0 = e0a + sb * EMAX
            ne_sb = jnp.minimum(e1 - sb_e0, EMAX)
            nch = (ne_sb + (C - 1)) // C
            nst = (ne_sb + (S - 1)) // S

            def stage(t, _):
                pltpu.sync_copy(srcp_hbm.at[pl.ds(sb_e0 + t * S, S)],
                                sbuf.at[pl.ds(t * S, S)])
                pltpu.sync_copy(dstp_hbm.at[pl.ds(sb_e0 + t * S, S)],
                                dbuf.at[pl.ds(t * S, S)])
                return 0

            lax.fori_loop(0, nst, stage, 0)

            def prefetch(cj, buf):
                @pl.when(cj < nch)
                def _():
                    pltpu.async_copy(x_hbm.at[sbuf.at[pl.ds(cj * C, C)]],
                                     gbuf.at[buf], sems[buf])

            def process(ci, buf):
                @pl.when(ci < nch)
                def _():
                    pltpu.make_async_copy(
                        x_hbm.at[sbuf.at[pl.ds(ci * C, C)]],
                        gbuf.at[buf], sems[buf]).wait()
                    accumulate(ci, buf)

            prefetch(0, 0)

            def pair(i2, _):
                i = i2 * 2
                prefetch(i + 1, 1)
                process(i, 0)
                prefetch(i + 2, 0)
                process(i + 1, 1)
                return 0

            lax.fori_loop(0, (nch + 1) // 2, pair, 0)
            return 0

        lax.fori_loop(0, nsb, superblock, 0)

        pltpu.sync_copy(mask_hbm.at[pl.ds(nbase, NB)], mask_v)
        pltpu.sync_copy(inv_hbm.at[pl.ds(nbase, NB)], inv_v)
        for rc in range(NB // RC):
            r0 = nbase + rc * RC
            pltpu.sync_copy(x_hbm.at[pl.ds(r0, RC)], gbuf.at[0, pl.ds(0, RC)])

            def fgroup(g, _):
                mv16 = mask_v[pl.ds(rc * RC + g * 16, 16)]
                iv16 = inv_v[pl.ds(rc * RC + g * 16, 16)]
                for u in range(16):
                    r = g * 16 + u
                    mv = jnp.full((16,), mv16[u], jnp.float32)
                    iv = jnp.full((16,), iv16[u], jnp.float32)
                    for j in range(D // 16):
                        xr = gbuf[0, r, pl.ds(j * 16, 16)]
                        ar = acc[rc * RC + r, pl.ds(j * 16, 16)]
                        gbuf[1, r, pl.ds(j * 16, 16)] = mv * xr - iv * ar
                return 0

            lax.fori_loop(0, RC // 16, fgroup, 0)
            pltpu.sync_copy(gbuf.at[1, pl.ds(0, RC)], y_hbm.at[pl.ds(r0, RC)])

    return step(x, srcp, dstp, estp, maskp, invp)


_BLK = 1024


def _finale_body(data_ref, *rest):
    u_refs = rest[:DEPTH]
    (wt0, wt1, wt2, b0, b1, b2, wts, bs, wtt, bt, out_ref) = rest[DEPTH:]
    data = data_ref[...]
    us = [r[...] for r in u_refs]
    wts_ = [wt0[...], wt1[...], wt2[...]]
    bs_ = [b0[...], b1[...], b2[...]]

    outs = []
    for s, t in enumerate(TELEPORTS):
        coeff = [t * (1.0 - t) ** k for k in range(DEPTH)] + [(1.0 - t) ** DEPTH]
        A = coeff[0] * data
        for k in range(DEPTH):
            A = A + coeff[k + 1] * us[k]
        proj = jax.lax.dot(A, wts_[s], precision="highest",
                           preferred_element_type=jnp.float32)
        outs.append(jax.nn.relu(proj + t * bs_[s]))

    sa = jax.lax.dot(data, wts[...], precision="highest",
                     preferred_element_type=jnp.float32) + bs[...]
    logits = []
    for s in range(3):
        tgt = jax.lax.dot(outs[s], wtt[...], precision="highest",
                          preferred_element_type=jnp.float32) + bt[...]
        logits.append(jnp.sum(tgt * sa, axis=1, keepdims=True))
    m = jnp.maximum(jnp.maximum(logits[0], logits[1]), logits[2])
    es = [jnp.exp(l - m) for l in logits]
    z = es[0] + es[1] + es[2]
    acc = es[0] * outs[0] + es[1] * outs[1] + es[2] * outs[2]
    out_ref[...] = acc / z


def _finale(data_p, us, W0, b0, W1, b1, W2, b2, W_src, b_src, W_tgt, b_tgt):
    grid = (N_PAD // _BLK,)
    row_spec = pl.BlockSpec((_BLK, D), lambda i: (i, 0))
    mat_spec = pl.BlockSpec((D, D), lambda i: (0, 0))
    vec_spec = pl.BlockSpec((1, D), lambda i: (0, 0))
    args = (
        [data_p] + list(us)
        + [W0.T, W1.T, W2.T,
           b0.reshape(1, D), b1.reshape(1, D), b2.reshape(1, D),
           W_src.T, b_src.reshape(1, D), W_tgt.T, b_tgt.reshape(1, D)]
    )
    in_specs = [row_spec] * (1 + DEPTH) + [mat_spec] * 3 + [vec_spec] * 3 \
        + [mat_spec, vec_spec, mat_spec, vec_spec]
    return pl.pallas_call(
        _finale_body,
        grid=grid,
        in_specs=in_specs,
        out_specs=row_spec,
        out_shape=jax.ShapeDtypeStruct((N_PAD, D), jnp.float32),
    )(*args)


def kernel(data, edge_index, W_src, b_src, W_tgt, b_tgt, W0, b0, W1, b1, W2, b2):
    n = data.shape[0]
    src = edge_index[0].astype(jnp.int32)
    dst = edge_index[1].astype(jnp.int32)

    deg = jnp.bincount(dst, length=n)
    degf = deg.astype(jnp.float32)
    maskf = (deg > 0).astype(jnp.float32)
    invdeg = jnp.where(deg > 0, 1.0 / jnp.maximum(degf, 1.0), 0.0)

    x0 = jnp.pad(data, ((0, N_PAD - n), (0, 0)))
    maskp = jnp.pad(maskf, (0, N_PAD - n))
    invp = jnp.pad(invdeg, (0, N_PAD - n))

    srcp = jnp.pad(src, (0, EP - E))
    dstp = jnp.pad(dst, (0, EP - E), constant_values=N_PAD)
    bases = jnp.arange(NW + 1, dtype=jnp.int32) * NB
    estarts = jnp.searchsorted(dst, bases, side="left").astype(jnp.int32)
    estp = jnp.pad(estarts, (0, 48 - (NW + 1)))

    us = []
    x = x0
    for _ in range(DEPTH):
        x = _sc_step_call(x, srcp, dstp, estp, maskp, invp)
        us.append(x)

    out = _finale(x0, us, W0, b0, W1, b1, W2, b2, W_src, b_src, W_tgt, b_tgt)
    return out[:n]


# speedup vs baseline: 1.4238x; 1.4238x over previous
import functools

import jax
import jax.numpy as jnp
from jax import lax
from jax.experimental import pallas as pl
from jax.experimental.pallas import tpu as pltpu
from jax.experimental.pallas import tpu_sc as plsc

N = 10000
E = 320000
D = 128
DEPTH = 10
TELEPORTS = (0.1, 0.2, 0.3)

NW = 32
N_PAD = 10240
NB = N_PAD // NW
C = 128
S = 4096
EMAX = 16384
EP = E + S + C
RC = 64


def _sc_step_call(x, srcp, dstp, estp, maskp, invp):
    mesh = plsc.VectorSubcoreMesh(
        core_axis_name="c", subcore_axis_name="s", num_cores=2, num_subcores=16
    )

    @functools.partial(
        pl.kernel,
        out_type=jax.ShapeDtypeStruct((N_PAD, D), jnp.float32),
        mesh=mesh,
        compiler_params=pltpu.CompilerParams(needs_layout_passes=False),
        scratch_types=[
            pltpu.VMEM((NB + 8, D), jnp.float32),
            pltpu.VMEM((2, C, D), jnp.float32),
            pltpu.VMEM((EMAX,), jnp.int32),
            pltpu.VMEM((EMAX + 16,), jnp.int32),
            pltpu.VMEM((48,), jnp.int32),
            pltpu.VMEM((NB,), jnp.float32),
            pltpu.VMEM((NB,), jnp.float32),
            pltpu.SemaphoreType.DMA,
            pltpu.SemaphoreType.DMA,
        ],
    )
    def step(x_hbm, srcp_hbm, dstp_hbm, est_hbm, mask_hbm, inv_hbm, y_hbm,
             acc, gbuf, sbuf, dbuf, est_v, mask_v, inv_v, sem0, sem1):
        cid = lax.axis_index("c")
        sid = lax.axis_index("s")
        wid = cid * 16 + sid
        nbase = wid * NB

        zero16 = jnp.zeros((16,), jnp.float32)

        def zrow(r, _):
            for j in range(D // 16):
                acc[r, pl.ds(j * 16, 16)] = zero16
            return 0

        lax.fori_loop(0, NB + 8, zrow, 0)

        pltpu.sync_copy(est_hbm, est_v)
        ew = est_v[pl.ds(wid, 16)]
        e0 = ew[0]
        e1 = ew[1]
        e0a = (e0 // 8) * 8
        nsb = (e1 - e0a + (EMAX - 1)) // EMAX

        cols = [lax.iota(jnp.int32, 16) + 16 * j for j in range(D // 16)]
        sems = (sem0, sem1)

        def accumulate(ci, buf):
            def egroup(g, _):
                dv = dbuf[pl.ds(ci * C + g * 16, 16)]
                locv16 = dv - nbase
                bad = jnp.logical_or(locv16 < 0, locv16 >= NB)
                locv16 = jnp.where(bad, NB, locv16)
                lo0 = locv16[0]
                lo15 = locv16[15]

                @pl.when(lo0 == lo15)
                def _():
                    for j in range(D // 16):
                        t = gbuf[buf, g * 16, pl.ds(j * 16, 16)]
                        for u in range(1, 16):
                            t = t + gbuf[buf, g * 16 + u, pl.ds(j * 16, 16)]
                        plsc.addupdate(acc.at[lo0, pl.ds(j * 16, 16)], t)

                @pl.when(lo0 != lo15)
                def _():
                    dv2 = dbuf[pl.ds(ci * C + g * 16 + 1, 16)]
                    locb = dv2 - nbase
                    badb = jnp.logical_or(locb < 0, locb >= NB)
                    locb = jnp.where(badb, NB, locb)
                    lane = lax.iota(jnp.int32, 16)
                    bm = jnp.logical_and(locv16 != locb, lane < 15)
                    bmi = jnp.where(bm, 1, 0)
                    nb_ = jnp.sum(bmi)
                    p0 = jnp.sum(jnp.where(bm, lane, 0))

                    @pl.when(nb_ == 1)
                    def _():
                        for j in range(D // 16):
                            pre = []
                            t = gbuf[buf, g * 16, pl.ds(j * 16, 16)]
                            pre.append(t)
                            for u in range(1, 16):
                                t = t + gbuf[buf, g * 16 + u, pl.ds(j * 16, 16)]
                                pre.append(t)
                            ta = pre[0]
                            for u in range(1, 15):
                                ta = jnp.where(p0 == u, pre[u], ta)
                            tb = pre[15] - ta
                            plsc.addupdate(acc.at[lo0, pl.ds(j * 16, 16)], ta)
                            plsc.addupdate(acc.at[lo15, pl.ds(j * 16, 16)], tb)

                    @pl.when(nb_ != 1)
                    def _():
                        for u in range(16):
                            loc = locv16[u]
                            e = g * 16 + u
                            for j in range(D // 16):
                                v = gbuf[buf, e, pl.ds(j * 16, 16)]
                                plsc.addupdate(acc.at[loc, pl.ds(j * 16, 16)], v)
                return 0

            lax.fori_loop(0, C // 16, egroup, 0)

        def superblock(sb, _):
            sb_e0 = e0a + sb * EMAX
            ne_sb = jnp.minimum(e1 - sb_e0, EMAX)
            nch = (ne_sb + (C - 1)) // C
            nst = (ne_sb + (S - 1)) // S

            def stage(t, _):
                pltpu.sync_copy(srcp_hbm.at[pl.ds(sb_e0 + t * S, S)],
                                sbuf.at[pl.ds(t * S, S)])
                pltpu.sync_copy(dstp_hbm.at[pl.ds(sb_e0 + t * S, S)],
                                dbuf.at[pl.ds(t * S, S)])
                return 0

            lax.fori_loop(0, nst, stage, 0)

            def prefetch(cj, buf):
                @pl.when(cj < nch)
                def _():
                    pltpu.async_copy(x_hbm.at[sbuf.at[pl.ds(cj * C, C)]],
                                     gbuf.at[buf], sems[buf])

            def process(ci, buf):
                @pl.when(ci < nch)
                def _():
                    pltpu.make_async_copy(
                        x_hbm.at[sbuf.at[pl.ds(ci * C, C)]],
                        gbuf.at[buf], sems[buf]).wait()
                    accumulate(ci, buf)

            prefetch(0, 0)

            def pair(i2, _):
                i = i2 * 2
                prefetch(i + 1, 1)
                process(i, 0)
                prefetch(i + 2, 0)
                process(i + 1, 1)
                return 0

            lax.fori_loop(0, (nch + 1) // 2, pair, 0)
            return 0

        lax.fori_loop(0, nsb, superblock, 0)

        pltpu.sync_copy(mask_hbm.at[pl.ds(nbase, NB)], mask_v)
        pltpu.sync_copy(inv_hbm.at[pl.ds(nbase, NB)], inv_v)
        for rc in range(NB // RC):
            r0 = nbase + rc * RC
            pltpu.sync_copy(x_hbm.at[pl.ds(r0, RC)], gbuf.at[0, pl.ds(0, RC)])

            def fgroup(g, _):
                mv16 = mask_v[pl.ds(rc * RC + g * 16, 16)]
                iv16 = inv_v[pl.ds(rc * RC + g * 16, 16)]
                for u in range(16):
                    r = g * 16 + u
                    mv = jnp.full((16,), mv16[u], jnp.float32)
                    iv = jnp.full((16,), iv16[u], jnp.float32)
                    for j in range(D // 16):
                        xr = gbuf[0, r, pl.ds(j * 16, 16)]
                        ar = acc[rc * RC + r, pl.ds(j * 16, 16)]
                        gbuf[1, r, pl.ds(j * 16, 16)] = mv * xr - iv * ar
                return 0

            lax.fori_loop(0, RC // 16, fgroup, 0)
            pltpu.sync_copy(gbuf.at[1, pl.ds(0, RC)], y_hbm.at[pl.ds(r0, RC)])

    return step(x, srcp, dstp, estp, maskp, invp)


_BLK = 1024


def _finale_body(data_ref, *rest):
    u_refs = rest[:DEPTH]
    (wt0, wt1, wt2, b0, b1, b2, wts, bs, wtt, bt, out_ref) = rest[DEPTH:]
    data = data_ref[...]
    us = [r[...] for r in u_refs]
    wts_ = [wt0[...], wt1[...], wt2[...]]
    bs_ = [b0[...], b1[...], b2[...]]

    outs = []
    for s, t in enumerate(TELEPORTS):
        coeff = [t * (1.0 - t) ** k for k in range(DEPTH)] + [(1.0 - t) ** DEPTH]
        A = coeff[0] * data
        for k in range(DEPTH):
            A = A + coeff[k + 1] * us[k]
        proj = jax.lax.dot(A, wts_[s], precision="highest",
                           preferred_element_type=jnp.float32)
        outs.append(jax.nn.relu(proj + t * bs_[s]))

    sa = jax.lax.dot(data, wts[...], precision="highest",
                     preferred_element_type=jnp.float32) + bs[...]
    logits = []
    for s in range(3):
        tgt = jax.lax.dot(outs[s], wtt[...], precision="highest",
                          preferred_element_type=jnp.float32) + bt[...]
        logits.append(jnp.sum(tgt * sa, axis=1, keepdims=True))
    m = jnp.maximum(jnp.maximum(logits[0], logits[1]), logits[2])
    es = [jnp.exp(l - m) for l in logits]
    z = es[0] + es[1] + es[2]
    acc = es[0] * outs[0] + es[1] * outs[1] + es[2] * outs[2]
    out_ref[...] = acc / z


def _finale(data_p, us, W0, b0, W1, b1, W2, b2, W_src, b_src, W_tgt, b_tgt):
    grid = (N_PAD // _BLK,)
    row_spec = pl.BlockSpec((_BLK, D), lambda i: (i, 0))
    mat_spec = pl.BlockSpec((D, D), lambda i: (0, 0))
    vec_spec = pl.BlockSpec((1, D), lambda i: (0, 0))
    args = (
        [data_p] + list(us)
        + [W0.T, W1.T, W2.T,
           b0.reshape(1, D), b1.reshape(1, D), b2.reshape(1, D),
           W_src.T, b_src.reshape(1, D), W_tgt.T, b_tgt.reshape(1, D)]
    )
    in_specs = [row_spec] * (1 + DEPTH) + [mat_spec] * 3 + [vec_spec] * 3 \
        + [mat_spec, vec_spec, mat_spec, vec_spec]
    return pl.pallas_call(
        _finale_body,
        grid=grid,
        in_specs=in_specs,
        out_specs=row_spec,
        out_shape=jax.ShapeDtypeStruct((N_PAD, D), jnp.float32),
    )(*args)


def kernel(data, edge_index, W_src, b_src, W_tgt, b_tgt, W0, b0, W1, b1, W2, b2):
    n = data.shape[0]
    src = edge_index[0].astype(jnp.int32)
    dst = edge_index[1].astype(jnp.int32)

    deg = jnp.bincount(dst, length=n)
    degf = deg.astype(jnp.float32)
    maskf = (deg > 0).astype(jnp.float32)
    invdeg = jnp.where(deg > 0, 1.0 / jnp.maximum(degf, 1.0), 0.0)

    x0 = jnp.pad(data, ((0, N_PAD - n), (0, 0)))
    maskp = jnp.pad(maskf, (0, N_PAD - n))
    invp = jnp.pad(invdeg, (0, N_PAD - n))

    srcp = jnp.pad(src, (0, EP - E))
    dstp = jnp.pad(dst, (0, EP - E), constant_values=N_PAD)
    bases = jnp.arange(NW + 1, dtype=jnp.int32) * NB
    estarts = jnp.searchsorted(dst, bases, side="left").astype(jnp.int32)
    estp = jnp.pad(estarts, (0, 48 - (NW + 1)))

    us = []
    x = x0
    for _ in range(DEPTH):
        x = _sc_step_call(x, srcp, dstp, estp, maskp, invp)
        us.append(x)

    out = _finale(x0, us, W0, b0, W1, b1, W2, b2, W_src, b_src, W_tgt, b_tgt)
    return out[:n]


# speedup vs baseline: 1.5270x; 1.0725x over previous
import functools

import jax
import jax.numpy as jnp
from jax import lax
from jax.experimental import pallas as pl
from jax.experimental.pallas import tpu as pltpu
from jax.experimental.pallas import tpu_sc as plsc

N = 10000
E = 320000
D = 128
DEPTH = 10
TELEPORTS = (0.1, 0.2, 0.3)

NW = 32
N_PAD = 10240
NB = N_PAD // NW
C = 128
S = 4096
EMAX = 16384
EP = E + S + C
RC = 64


def _sc_step_call(x, srcp, dstp, estp, maskp, invp):
    mesh = plsc.VectorSubcoreMesh(
        core_axis_name="c", subcore_axis_name="s", num_cores=2, num_subcores=16
    )

    @functools.partial(
        pl.kernel,
        out_type=jax.ShapeDtypeStruct((N_PAD, D), jnp.float32),
        mesh=mesh,
        compiler_params=pltpu.CompilerParams(needs_layout_passes=False),
        scratch_types=[
            pltpu.VMEM((NB + 8, D), jnp.float32),
            pltpu.VMEM((2, C, D), jnp.float32),
            pltpu.VMEM((EMAX,), jnp.int32),
            pltpu.VMEM((EMAX + 16,), jnp.int32),
            pltpu.VMEM((48,), jnp.int32),
            pltpu.VMEM((NB,), jnp.float32),
            pltpu.VMEM((NB,), jnp.float32),
            pltpu.SemaphoreType.DMA,
            pltpu.SemaphoreType.DMA,
        ],
    )
    def step(x_hbm, srcp_hbm, dstp_hbm, est_hbm, mask_hbm, inv_hbm, y_hbm,
             acc, gbuf, sbuf, dbuf, est_v, mask_v, inv_v, sem0, sem1):
        cid = lax.axis_index("c")
        sid = lax.axis_index("s")
        wid = cid * 16 + sid
        nbase = wid * NB

        zero16 = jnp.zeros((16,), jnp.float32)

        def zrow(r, _):
            for j in range(D // 16):
                acc[r, pl.ds(j * 16, 16)] = zero16
            return 0

        lax.fori_loop(0, NB + 8, zrow, 0)

        pltpu.sync_copy(est_hbm, est_v)
        ew = est_v[pl.ds(wid, 16)]
        e0 = ew[0]
        e1 = ew[1]
        e0a = (e0 // 8) * 8
        nsb = (e1 - e0a + (EMAX - 1)) // EMAX

        cols = [lax.iota(jnp.int32, 16) + 16 * j for j in range(D // 16)]
        sems = (sem0, sem1)

        def accumulate(ci, buf):
            def egroup(g, _):
                dv = dbuf[pl.ds(ci * C + g * 16, 16)]
                locv16 = dv - nbase
                bad = jnp.logical_or(locv16 < 0, locv16 >= NB)
                locv16 = jnp.where(bad, NB, locv16)
                lo0 = locv16[0]
                lo15 = locv16[15]

                def tree(vals):
                    while len(vals) > 1:
                        vals = [a + b for a, b in zip(vals[::2], vals[1::2])]
                    return vals[0]

                @pl.when(lo0 == lo15)
                def _():
                    for j in range(D // 16):
                        rows = [gbuf[buf, g * 16 + u, pl.ds(j * 16, 16)]
                                for u in range(16)]
                        plsc.addupdate(acc.at[lo0, pl.ds(j * 16, 16)], tree(rows))

                @pl.when(lo0 != lo15)
                def _():
                    dv2 = dbuf[pl.ds(ci * C + g * 16 + 1, 16)]
                    locb = dv2 - nbase
                    badb = jnp.logical_or(locb < 0, locb >= NB)
                    locb = jnp.where(badb, NB, locb)
                    lane = lax.iota(jnp.int32, 16)
                    bm = jnp.logical_and(locv16 != locb, lane < 15)
                    bmi = jnp.where(bm, 1, 0)
                    nb_ = jnp.sum(bmi)
                    p0 = jnp.sum(jnp.where(bm, lane, 0))

                    @pl.when(nb_ == 1)
                    def _():
                        zerov = jnp.zeros((16,), jnp.float32)
                        sels = [p0 >= u for u in range(16)]
                        for j in range(D // 16):
                            rows = [gbuf[buf, g * 16 + u, pl.ds(j * 16, 16)]
                                    for u in range(16)]
                            masked = [jnp.where(sels[u], rows[u], zerov)
                                      for u in range(16)]
                            ta = tree(masked)
                            tb = tree(rows) - ta
                            plsc.addupdate(acc.at[lo0, pl.ds(j * 16, 16)], ta)
                            plsc.addupdate(acc.at[lo15, pl.ds(j * 16, 16)], tb)

                    @pl.when(nb_ != 1)
                    def _():
                        for u in range(16):
                            loc = locv16[u]
                            e = g * 16 + u
                            for j in range(D // 16):
                                v = gbuf[buf, e, pl.ds(j * 16, 16)]
                                plsc.addupdate(acc.at[loc, pl.ds(j * 16, 16)], v)
                return 0

            lax.fori_loop(0, C // 16, egroup, 0)

        def superblock(sb, _):
            sb_e0 = e0a + sb * EMAX
            ne_sb = jnp.minimum(e1 - sb_e0, EMAX)
            nch = (ne_sb + (C - 1)) // C
            nst = (ne_sb + (S - 1)) // S

            def stage(t, _):
                pltpu.sync_copy(srcp_hbm.at[pl.ds(sb_e0 + t * S, S)],
                                sbuf.at[pl.ds(t * S, S)])
                pltpu.sync_copy(dstp_hbm.at[pl.ds(sb_e0 + t * S, S)],
                                dbuf.at[pl.ds(t * S, S)])
                return 0

            lax.fori_loop(0, nst, stage, 0)

            def prefetch(cj, buf):
                @pl.when(cj < nch)
                def _():
                    pltpu.async_copy(x_hbm.at[sbuf.at[pl.ds(cj * C, C)]],
                                     gbuf.at[buf], sems[buf])

            def process(ci, buf):
                @pl.when(ci < nch)
                def _():
                    pltpu.make_async_copy(
                        x_hbm.at[sbuf.at[pl.ds(ci * C, C)]],
                        gbuf.at[buf], sems[buf]).wait()
                    accumulate(ci, buf)

            prefetch(0, 0)

            def pair(i2, _):
                i = i2 * 2
                prefetch(i + 1, 1)
                process(i, 0)
                prefetch(i + 2, 0)
                process(i + 1, 1)
                return 0

            lax.fori_loop(0, (nch + 1) // 2, pair, 0)
            return 0

        lax.fori_loop(0, nsb, superblock, 0)

        pltpu.sync_copy(mask_hbm.at[pl.ds(nbase, NB)], mask_v)
        pltpu.sync_copy(inv_hbm.at[pl.ds(nbase, NB)], inv_v)
        for rc in range(NB // RC):
            r0 = nbase + rc * RC
            pltpu.sync_copy(x_hbm.at[pl.ds(r0, RC)], gbuf.at[0, pl.ds(0, RC)])

            def fgroup(g, _):
                mv16 = mask_v[pl.ds(rc * RC + g * 16, 16)]
                iv16 = inv_v[pl.ds(rc * RC + g * 16, 16)]
                for u in range(16):
                    r = g * 16 + u
                    mv = jnp.full((16,), mv16[u], jnp.float32)
                    iv = jnp.full((16,), iv16[u], jnp.float32)
                    for j in range(D // 16):
                        xr = gbuf[0, r, pl.ds(j * 16, 16)]
                        ar = acc[rc * RC + r, pl.ds(j * 16, 16)]
                        gbuf[1, r, pl.ds(j * 16, 16)] = mv * xr - iv * ar
                return 0

            lax.fori_loop(0, RC // 16, fgroup, 0)
            pltpu.sync_copy(gbuf.at[1, pl.ds(0, RC)], y_hbm.at[pl.ds(r0, RC)])

    return step(x, srcp, dstp, estp, maskp, invp)


_BLK = 1024


def _finale_body(data_ref, *rest):
    u_refs = rest[:DEPTH]
    (wt0, wt1, wt2, b0, b1, b2, wts, bs, wtt, bt, out_ref) = rest[DEPTH:]
    data = data_ref[...]
    us = [r[...] for r in u_refs]
    wts_ = [wt0[...], wt1[...], wt2[...]]
    bs_ = [b0[...], b1[...], b2[...]]

    outs = []
    for s, t in enumerate(TELEPORTS):
        coeff = [t * (1.0 - t) ** k for k in range(DEPTH)] + [(1.0 - t) ** DEPTH]
        A = coeff[0] * data
        for k in range(DEPTH):
            A = A + coeff[k + 1] * us[k]
        proj = jax.lax.dot(A, wts_[s], precision="highest",
                           preferred_element_type=jnp.float32)
        outs.append(jax.nn.relu(proj + t * bs_[s]))

    sa = jax.lax.dot(data, wts[...], precision="highest",
                     preferred_element_type=jnp.float32) + bs[...]
    logits = []
    for s in range(3):
        tgt = jax.lax.dot(outs[s], wtt[...], precision="highest",
                          preferred_element_type=jnp.float32) + bt[...]
        logits.append(jnp.sum(tgt * sa, axis=1, keepdims=True))
    m = jnp.maximum(jnp.maximum(logits[0], logits[1]), logits[2])
    es = [jnp.exp(l - m) for l in logits]
    z = es[0] + es[1] + es[2]
    acc = es[0] * outs[0] + es[1] * outs[1] + es[2] * outs[2]
    out_ref[...] = acc / z


def _finale(data_p, us, W0, b0, W1, b1, W2, b2, W_src, b_src, W_tgt, b_tgt):
    grid = (N_PAD // _BLK,)
    row_spec = pl.BlockSpec((_BLK, D), lambda i: (i, 0))
    mat_spec = pl.BlockSpec((D, D), lambda i: (0, 0))
    vec_spec = pl.BlockSpec((1, D), lambda i: (0, 0))
    args = (
        [data_p] + list(us)
        + [W0.T, W1.T, W2.T,
           b0.reshape(1, D), b1.reshape(1, D), b2.reshape(1, D),
           W_src.T, b_src.reshape(1, D), W_tgt.T, b_tgt.reshape(1, D)]
    )
    in_specs = [row_spec] * (1 + DEPTH) + [mat_spec] * 3 + [vec_spec] * 3 \
        + [mat_spec, vec_spec, mat_spec, vec_spec]
    return pl.pallas_call(
        _finale_body,
        grid=grid,
        in_specs=in_specs,
        out_specs=row_spec,
        out_shape=jax.ShapeDtypeStruct((N_PAD, D), jnp.float32),
    )(*args)


def kernel(data, edge_index, W_src, b_src, W_tgt, b_tgt, W0, b0, W1, b1, W2, b2):
    n = data.shape[0]
    src = edge_index[0].astype(jnp.int32)
    dst = edge_index[1].astype(jnp.int32)

    deg = jnp.bincount(dst, length=n)
    degf = deg.astype(jnp.float32)
    maskf = (deg > 0).astype(jnp.float32)
    invdeg = jnp.where(deg > 0, 1.0 / jnp.maximum(degf, 1.0), 0.0)

    x0 = jnp.pad(data, ((0, N_PAD - n), (0, 0)))
    maskp = jnp.pad(maskf, (0, N_PAD - n))
    invp = jnp.pad(invdeg, (0, N_PAD - n))

    srcp = jnp.pad(src, (0, EP - E))
    dstp = jnp.pad(dst, (0, EP - E), constant_values=N_PAD)
    bases = jnp.arange(NW + 1, dtype=jnp.int32) * NB
    estarts = jnp.searchsorted(dst, bases, side="left").astype(jnp.int32)
    estp = jnp.pad(estarts, (0, 48 - (NW + 1)))

    us = []
    x = x0
    for _ in range(DEPTH):
        x = _sc_step_call(x, srcp, dstp, estp, maskp, invp)
        us.append(x)

    out = _finale(x0, us, W0, b0, W1, b1, W2, b2, W_src, b_src, W_tgt, b_tgt)
    return out[:n]
